# baseline (speedup 1.0000x reference)
import functools

import jax
import jax.numpy as jnp
import numpy as np
from jax.experimental import pallas as pl
from jax.experimental.pallas import tpu as pltpu

BM = 256
BK = 256
NEG = -3.0e38


def _ceil_to(n, g):
    return ((n + g - 1) // g) * g


def _mm_body(a_ref, rs_ref, cs_ref, b_ref, c_ref, bias_ref, o_ref, acc_ref,
             *, nk, relu):
    k = pl.program_id(1)

    @pl.when(k == 0)
    def _init():
        acc_ref[...] = jnp.zeros_like(acc_ref)

    a = (rs_ref[...] * a_ref[...]) * cs_ref[...]
    acc_ref[...] += jnp.dot(a, b_ref[...], preferred_element_type=jnp.float32)

    @pl.when(k == nk - 1)
    def _fin():
        r = (acc_ref[...] + c_ref[...]) + bias_ref[...]
        o_ref[...] = jnp.maximum(r, 0.0) if relu else r


def _mm(a, b, rs=None, cs=None, c=None, bias=None, relu=False):
    m, kk = a.shape
    f = b.shape[1]
    if rs is None:
        rs = jnp.ones((m, 1), jnp.float32)
    if cs is None:
        cs = jnp.ones((1, kk), jnp.float32)
    if c is None:
        c = jnp.zeros((m, f), jnp.float32)
    if bias is None:
        bias = jnp.zeros((1, f), jnp.float32)
    bk = min(BK, kk)
    nk = kk // bk
    return pl.pallas_call(
        functools.partial(_mm_body, nk=nk, relu=relu),
        grid=(m // BM, nk),
        in_specs=[
            pl.BlockSpec((BM, bk), lambda i, k: (i, k)),
            pl.BlockSpec((BM, 1), lambda i, k: (i, 0)),
            pl.BlockSpec((1, bk), lambda i, k: (0, k)),
            pl.BlockSpec((bk, f), lambda i, k: (k, 0)),
            pl.BlockSpec((BM, f), lambda i, k: (i, 0)),
            pl.BlockSpec((1, f), lambda i, k: (0, 0)),
        ],
        out_specs=pl.BlockSpec((BM, f), lambda i, k: (i, 0)),
        out_shape=jax.ShapeDtypeStruct((m, f), jnp.float32),
        scratch_shapes=[pltpu.VMEM((BM, f), jnp.float32)],
        compiler_params=pltpu.CompilerParams(
            dimension_semantics=("parallel", "arbitrary")),
    )(a, rs, cs, b, c, bias)


def _a2a_body(a_ref, b_ref, o_ref, acc_ref, *, nk):
    k = pl.program_id(2)

    @pl.when(k == 0)
    def _init():
        acc_ref[...] = jnp.zeros_like(acc_ref)

    acc_ref[...] += jnp.dot(a_ref[...].astype(jnp.bfloat16),
                            b_ref[...].astype(jnp.bfloat16),
                            preferred_element_type=jnp.float32)

    @pl.when(k == nk - 1)
    def _fin():
        o_ref[...] = (acc_ref[...] > 0).astype(jnp.float32)


def _a2a(a):
    m = a.shape[0]
    nk = m // BK
    return pl.pallas_call(
        functools.partial(_a2a_body, nk=nk),
        grid=(m // BM, m // BM, nk),
        in_specs=[
            pl.BlockSpec((BM, BK), lambda i, j, k: (i, k)),
            pl.BlockSpec((BK, BM), lambda i, j, k: (k, j)),
        ],
        out_specs=pl.BlockSpec((BM, BM), lambda i, j, k: (i, j)),
        out_shape=jax.ShapeDtypeStruct((m, m), jnp.float32),
        scratch_shapes=[pltpu.VMEM((BM, BM), jnp.float32)],
        compiler_params=pltpu.CompilerParams(
            dimension_semantics=("parallel", "parallel", "arbitrary")),
    )(a, a)


def _dis_body(a_ref, o_ref):
    o_ref[...] = jnp.sum(a_ref[...], axis=1, keepdims=True)


def _dis(a):
    m, kk = a.shape
    d = pl.pallas_call(
        _dis_body,
        grid=(m // BM,),
        in_specs=[pl.BlockSpec((BM, kk), lambda i: (i, 0))],
        out_specs=pl.BlockSpec((BM, 1), lambda i: (i, 0)),
        out_shape=jax.ShapeDtypeStruct((m, 1), jnp.float32),
    )(a)
    return jnp.where(d > 0, 1.0 / jnp.sqrt(d), 0.0)


def _rank_body(s_ref, srow_ref, o_ref, *, m):
    i = pl.program_id(0)
    sc = s_ref[...]
    sr = srow_ref[...]
    gt = (sr > sc).astype(jnp.int32)
    jidx = jax.lax.broadcasted_iota(jnp.int32, (BM, m), 1)
    iidx = i * BM + jax.lax.broadcasted_iota(jnp.int32, (BM, m), 0)
    eqb = ((sr == sc) & (jidx < iidx)).astype(jnp.int32)
    o_ref[...] = jnp.sum(gt + eqb, axis=1, keepdims=True)


def _rank(s_col, s_row):
    m = s_col.shape[0]
    return pl.pallas_call(
        functools.partial(_rank_body, m=m),
        grid=(m // BM,),
        in_specs=[pl.BlockSpec((BM, 1), lambda i: (i, 0)),
                  pl.BlockSpec((1, m), lambda i: (0, 0))],
        out_specs=pl.BlockSpec((BM, 1), lambda i: (i, 0)),
        out_shape=jax.ShapeDtypeStruct((m, 1), jnp.int32),
    )(s_col, s_row)


def _perm_body(rank_ref, srow_ref, perm_ref, scale_ref, mask_ref, *, m, k):
    r0 = pl.program_id(0) * BM
    rr = r0 + jax.lax.broadcasted_iota(jnp.int32, (BM, m), 0)
    hit = rank_ref[...] == rr
    jidx = jax.lax.broadcasted_iota(jnp.int32, (BM, m), 1)
    perm = jnp.sum(jnp.where(hit, jidx, 0), axis=1, keepdims=True)
    sp = jnp.sum(jnp.where(hit, srow_ref[...], 0.0), axis=1, keepdims=True)
    valid = rr[:, :1] < k
    perm_ref[...] = jnp.where(valid, perm, 0)
    scale_ref[...] = jnp.where(valid, sp, 0.0)
    mask_ref[...] = jnp.where(valid, 1.0, 0.0)


def _perm_pack(rank_row, s_row, k, m_out):
    m = rank_row.shape[1]
    return pl.pallas_call(
        functools.partial(_perm_body, m=m, k=k),
        grid=(m_out // BM,),
        in_specs=[pl.BlockSpec((1, m), lambda i: (0, 0)),
                  pl.BlockSpec((1, m), lambda i: (0, 0))],
        out_specs=[pl.BlockSpec((BM, 1), lambda i: (i, 0))] * 3,
        out_shape=[jax.ShapeDtypeStruct((m_out, 1), jnp.int32),
                   jax.ShapeDtypeStruct((m_out, 1), jnp.float32),
                   jax.ShapeDtypeStruct((m_out, 1), jnp.float32)],
    )(rank_row, s_row)


def _inv_body(permrow_ref, inv_ref, mask_ref, *, mk, k):
    i0 = pl.program_id(0) * BM
    ii = i0 + jax.lax.broadcasted_iota(jnp.int32, (BM, mk), 0)
    rj = jax.lax.broadcasted_iota(jnp.int32, (BM, mk), 1)
    hit = (permrow_ref[...] == ii) & (rj < k)
    inv_ref[...] = jnp.sum(jnp.where(hit, rj, 0), axis=1, keepdims=True)
    mask_ref[...] = jnp.sum(hit.astype(jnp.float32), axis=1, keepdims=True)


def _inv_pack(perm_row, k, m_out):
    mk = perm_row.shape[1]
    return pl.pallas_call(
        functools.partial(_inv_body, mk=mk, k=k),
        grid=(m_out // BM,),
        in_specs=[pl.BlockSpec((1, mk), lambda i: (0, 0))],
        out_specs=[pl.BlockSpec((BM, 1), lambda i: (i, 0))] * 2,
        out_shape=[jax.ShapeDtypeStruct((m_out, 1), jnp.int32),
                   jax.ShapeDtypeStruct((m_out, 1), jnp.float32)],
    )(perm_row)


def _gather_body(idx_ref, src_ref, scale_ref, o_ref):
    del idx_ref
    o_ref[...] = src_ref[...] * scale_ref[...]


def _gather_rows(src, idx, scale):
    ms, w = src.shape
    mo = idx.shape[0]
    src3 = src.reshape(ms, 1, w)
    scale3 = scale.reshape(mo, 1, 1)
    out = pl.pallas_call(
        _gather_body,
        grid_spec=pltpu.PrefetchScalarGridSpec(
            num_scalar_prefetch=1,
            grid=(mo,),
            in_specs=[
                pl.BlockSpec((1, 1, w), lambda r, idx: (idx[r], 0, 0)),
                pl.BlockSpec((1, 1, 1), lambda r, idx: (r, 0, 0)),
            ],
            out_specs=pl.BlockSpec((1, 1, w), lambda r, idx: (r, 0, 0)),
        ),
        out_shape=jax.ShapeDtypeStruct((mo, 1, w), jnp.float32),
    )(idx, src3, scale3)
    return out.reshape(mo, w)


def _tr_body(a_ref, o_ref):
    o_ref[...] = a_ref[...].T


def _transpose(a):
    m, n = a.shape
    return pl.pallas_call(
        _tr_body,
        grid=(m // BM, n // BM),
        in_specs=[pl.BlockSpec((BM, BM), lambda i, j: (i, j))],
        out_specs=pl.BlockSpec((BM, BM), lambda i, j: (j, i)),
        out_shape=jax.ShapeDtypeStruct((n, m), jnp.float32),
    )(a)


def _pad2(a, m, n):
    return jnp.pad(a, ((0, m - a.shape[0]), (0, n - a.shape[1])))


def _gcn(a_pad, dis_col, dis_row, h, w_pad, b_pad, relu=False):
    hw = _mm(h, w_pad)
    return _mm(a_pad, hw, rs=dis_col, cs=dis_row, bias=b_pad, relu=relu)


def _sag_scores(a_pad, h, wr_pad, wn_pad, bs_pad, n_true):
    hwn = _mm(h, wn_pad)
    hwr = _mm(h, wr_pad)
    s = _mm(a_pad, hwn, c=hwr, bias=bs_pad)
    s_col = s[:, :1]
    if s_col.shape[0] > n_true:
        rowmask = (jnp.arange(s_col.shape[0]) < n_true)[:, None]
        s_col = jnp.where(rowmask, s_col, NEG)
    return s_col


def _pool_adj(a_pad, perm, mask):
    g1 = _gather_rows(a_pad, perm, mask)
    g1t = _transpose(g1)
    g2 = _gather_rows(g1t, perm, mask)
    return g2


def kernel(x, edge_index, y, batch, W1, b1, Wr1, Wn1, bs1, W2, b2, Wr2, Wn2,
           bs2, W3, b3, Wr3, Wn3, bs3, Wu0, bu0, Wu1, bu1, Wu2, bu2):
    del y
    n0, f_in = x.shape
    h = W1.shape[1]
    hp = _ceil_to(h, 128)
    fp = _ceil_to(f_in, 128)
    n0p = _ceil_to(n0, BM)

    k1 = int(np.ceil(0.8 * n0))
    k2 = int(np.ceil(0.8 * k1))
    k3 = int(np.ceil(0.8 * k2))
    n1p, n2p, n3p = _ceil_to(k1, BM), _ceil_to(k2, BM), _ceil_to(k3, BM)

    w1p = _pad2(W1, fp, hp)
    b1p = _pad2(b1[None, :], 1, hp)
    w2p, b2p = _pad2(W2, hp, hp), _pad2(b2[None, :], 1, hp)
    w3p, b3p = _pad2(W3, hp, hp), _pad2(b3[None, :], 1, hp)
    wr1p, wn1p = _pad2(Wr1, hp, hp), _pad2(Wn1, hp, hp)
    wr2p, wn2p = _pad2(Wr2, hp, hp), _pad2(Wn2, hp, hp)
    wr3p, wn3p = _pad2(Wr3, hp, hp), _pad2(Wn3, hp, hp)
    bs1p = _pad2(bs1[None, :], 1, hp)
    bs2p = _pad2(bs2[None, :], 1, hp)
    bs3p = _pad2(bs3[None, :], 1, hp)
    wu0p, bu0p = _pad2(Wu0, hp, hp), _pad2(bu0[None, :], 1, hp)
    wu1p, bu1p = _pad2(Wu1, hp, hp), _pad2(bu1[None, :], 1, hp)
    wu2p, bu2p = _pad2(Wu2, hp, fp), _pad2(bu2[None, :], 1, fp)

    xp = _pad2(x, n0p, fp)

    ar = jnp.arange(n0)
    a0 = jnp.zeros((n0p, n0p), jnp.float32)
    a0 = a0.at[edge_index[1], edge_index[0]].set(1.0)
    a0 = a0.at[ar, ar].set(1.0)

    dis0 = _dis(a0)
    dis0r = dis0.reshape(1, -1)
    x1 = _gcn(a0, dis0, dis0r, xp, w1p, b1p)
    s1 = _sag_scores(a0, x1, wr1p, wn1p, bs1p, n0)
    rank1 = _rank(s1, s1.reshape(1, -1))
    perm1, sp1, mask1 = _perm_pack(rank1.reshape(1, -1),
                                    s1.reshape(1, -1), k1, n1p)
    scale1 = jnp.tanh(sp1) * mask1
    perm1f = perm1.reshape(-1)
    x1p = _gather_rows(x1, perm1f, scale1)
    a1t = _pool_adj(a0, perm1f, mask1)
    a1a = _transpose(_a2a(a1t))

    dis1 = _dis(a1a)
    dis1r = dis1.reshape(1, -1)
    x2 = _gcn(a1a, dis1, dis1r, x1p, w2p, b2p)
    s2 = _sag_scores(a1a, x2, wr2p, wn2p, bs2p, k1)
    rank2 = _rank(s2, s2.reshape(1, -1))
    perm2, sp2, mask2 = _perm_pack(rank2.reshape(1, -1),
                                    s2.reshape(1, -1), k2, n2p)
    scale2 = jnp.tanh(sp2) * mask2
    perm2f = perm2.reshape(-1)
    x2p = _gather_rows(x2, perm2f, scale2)
    a2t = _pool_adj(a1a, perm2f, mask2)
    a2a_ = _transpose(_a2a(a2t))

    dis2 = _dis(a2a_)
    dis2r = dis2.reshape(1, -1)
    x3 = _gcn(a2a_, dis2, dis2r, x2p, w3p, b3p)
    s3 = _sag_scores(a2a_, x3, wr3p, wn3p, bs3p, k2)
    rank3 = _rank(s3, s3.reshape(1, -1))
    perm3, sp3, mask3 = _perm_pack(rank3.reshape(1, -1),
                                    s3.reshape(1, -1), k3, n3p)
    scale3 = jnp.tanh(sp3) * mask3
    perm3f = perm3.reshape(-1)
    x3p = _gather_rows(x3, perm3f, scale3)
    a3 = _transpose(_pool_adj(a2a_, perm3f, mask3))

    z = x3p
    inv3, um3 = _inv_pack(perm3.reshape(1, -1), k3, n2p)
    up = _gather_rows(z, inv3.reshape(-1), um3)
    z = _gcn(a2a_, dis2, dis2r, up, wu0p, bu0p, relu=True)

    inv2, um2 = _inv_pack(perm2.reshape(1, -1), k2, n1p)
    up = _gather_rows(z, inv2.reshape(-1), um2)
    z = _gcn(a1a, dis1, dis1r, up, wu1p, bu1p, relu=True)

    inv1, um1 = _inv_pack(perm1.reshape(1, -1), k1, n0p)
    up = _gather_rows(z, inv1.reshape(-1), um1)
    z = _gcn(a0, dis0, dis0r, up, wu2p, bu2p)

    z_out = z[:n0, :f_in]
    x3p_out = x3p[:k3, :h]
    a3_out = a3[:k3, :k3]
    bb3_out = jnp.zeros((k3,), batch.dtype)
    return z_out, x3p_out, a3_out, bb3_out
